# Initial kernel scaffold; baseline (speedup 1.0000x reference)
import jax
import jax.numpy as jnp
from jax.experimental import pallas as pl


def kernel(z, edge_index, Wl1, Wr1, b1, Wl2, Wr2, b2, Wres1, bres1, Wlf, Wrf, bf, Wres2, bres2):
    raise NotImplementedError("write your pallas kernel here")


# speedup vs baseline: 6.5217x; 6.5217x over previous
import functools

import jax
import jax.numpy as jnp
from jax import lax
from jax.experimental import pallas as pl
from jax.experimental.pallas import tpu as pltpu
from jax.experimental.pallas import tpu_sc as plsc

N_NODES = 10000
N_EDGES = 320000
D = 128

NC = 2
NS = 16
NW = NC * NS
EB = 128
NB_TOTAL = N_EDGES // EB

ROWS_PER_TILE = N_NODES // NS
STRIPE = 624
TAIL0 = STRIPE * NS
TAILN = N_NODES - TAIL0


def _make_sc_aggregate(with_deg: bool):
    mesh = plsc.VectorSubcoreMesh(core_axis_name="c", subcore_axis_name="s")
    agg_type = jax.ShapeDtypeStruct((NC, N_NODES, D), jnp.float32)
    if with_deg:
        out_type = [agg_type, jax.ShapeDtypeStruct((NC * N_NODES,), jnp.float32)]
    else:
        out_type = agg_type
    scratch_types = [
        pltpu.VMEM((EB,), jnp.int32),
        pltpu.VMEM((EB,), jnp.int32),
        pltpu.VMEM((EB, D), jnp.float32),
        pltpu.VMEM_SHARED((N_NODES, D), jnp.float32),
    ]
    if with_deg:
        scratch_types += [
            pltpu.VMEM((EB,), jnp.float32),
            pltpu.VMEM((STRIPE,), jnp.float32),
            pltpu.VMEM_SHARED((N_NODES,), jnp.float32),
        ]

    chunks = []
    roff = 0
    while roff < STRIPE:
        rlen = min(EB, STRIPE - roff)
        chunks.append((roff, rlen))
        roff += rlen

    def body(x_hbm, src_hbm, dst_hbm, *refs):
        if with_deg:
            agg_out, deg_out, src_v, dst_v, rows_v, acc_sh, ones_v, dv_v, deg_sh = refs
        else:
            agg_out, src_v, dst_v, rows_v, acc_sh = refs
        c = lax.axis_index("c")
        s = lax.axis_index("s")
        tile = c * NS + s

        zero16 = jnp.zeros((16,), jnp.float32)

        def zfill(i, carry):
            rows_v[i // 8, pl.ds((i % 8) * 16, 16)] = zero16
            return carry

        lax.fori_loop(0, EB * (D // 16), zfill, 0)

        r0 = s * STRIPE
        for roff, rlen in chunks:
            pltpu.sync_copy(rows_v.at[pl.ds(0, rlen)],
                            acc_sh.at[pl.ds(r0 + roff, rlen)])
        if with_deg:
            for j in range(EB // 16):
                ones_v[pl.ds(j * 16, 16)] = jnp.full((16,), 1.0, jnp.float32)
            for j in range(STRIPE // 16):
                dv_v[pl.ds(j * 16, 16)] = zero16
            pltpu.sync_copy(dv_v, deg_sh.at[pl.ds(r0, STRIPE)])

        @pl.when(s == NS - 1)
        def _():
            pltpu.sync_copy(rows_v.at[pl.ds(0, TAILN)], acc_sh.at[pl.ds(TAIL0, TAILN)])

        if with_deg:
            @pl.when(s == NS - 1)
            def _():
                pltpu.sync_copy(dv_v.at[pl.ds(0, TAILN)], deg_sh.at[pl.ds(TAIL0, TAILN)])

        plsc.subcore_barrier()

        lo = (tile * NB_TOTAL) // NW
        hi = ((tile + 1) * NB_TOTAL) // NW

        def batch(i, carry):
            off = i * EB
            pltpu.sync_copy(src_hbm.at[pl.ds(off, EB)], src_v)
            pltpu.sync_copy(dst_hbm.at[pl.ds(off, EB)], dst_v)
            pltpu.sync_copy(x_hbm.at[src_v], rows_v)
            pltpu.sync_copy(rows_v, acc_sh.at[dst_v], add=True)
            if with_deg:
                pltpu.sync_copy(ones_v, deg_sh.at[dst_v], add=True)
            return carry

        lax.fori_loop(lo, hi, batch, 0)
        plsc.subcore_barrier()

        for roff, rlen in chunks:
            pltpu.sync_copy(acc_sh.at[pl.ds(r0 + roff, rlen)],
                            rows_v.at[pl.ds(0, rlen)])
            pltpu.sync_copy(rows_v.at[pl.ds(0, rlen)],
                            agg_out.at[c, pl.ds(r0 + roff, rlen)])
        if with_deg:
            pltpu.sync_copy(deg_sh.at[pl.ds(r0, STRIPE)], dv_v)
            pltpu.sync_copy(dv_v, deg_out.at[pl.ds(c * N_NODES + r0, STRIPE)])

        @pl.when(s == NS - 1)
        def _():
            pltpu.sync_copy(acc_sh.at[pl.ds(TAIL0, TAILN)], rows_v.at[pl.ds(0, TAILN)])
            pltpu.sync_copy(rows_v.at[pl.ds(0, TAILN)], agg_out.at[c, pl.ds(TAIL0, TAILN)])

        if with_deg:
            @pl.when(s == NS - 1)
            def _():
                pltpu.sync_copy(deg_sh.at[pl.ds(TAIL0, TAILN)], dv_v.at[pl.ds(0, TAILN)])
                pltpu.sync_copy(dv_v.at[pl.ds(0, TAILN)],
                                deg_out.at[pl.ds(c * N_NODES + TAIL0, TAILN)])

    return pl.kernel(body, mesh=mesh, out_type=out_type, scratch_types=scratch_types)


def _make_epilogue(relu: bool, residual: bool, sigmoid4: bool):
    R = 1000

    def body(*refs):
        if residual:
            agg_ref, d0, d1, h_ref, z_ref, wl, wr, b, wres, bres, out_ref = refs
        else:
            agg_ref, d0, d1, h_ref, wl, wr, b, out_ref = refs
        agg = agg_ref[0] + agg_ref[1]
        deg = jnp.maximum(d0[...] + d1[...], 1.0)
        mean = agg / deg
        acc = jnp.dot(mean, wl[...], preferred_element_type=jnp.float32)
        acc = acc + jnp.dot(h_ref[...], wr[...], preferred_element_type=jnp.float32)
        acc = acc + b[...]
        if relu:
            acc = jnp.maximum(acc, 0.0)
        if residual:
            acc = acc + jnp.dot(z_ref[...], wres[...], preferred_element_type=jnp.float32)
            acc = acc + bres[...]
        if sigmoid4:
            col = lax.broadcasted_iota(jnp.int32, acc.shape, 1)
            sig = 1.0 / (1.0 + jnp.exp(-acc))
            acc = jnp.where(col < 4, sig, acc)
        out_ref[...] = acc

    in_specs = [
        pl.BlockSpec((NC, R, D), lambda i: (0, i, 0)),
        pl.BlockSpec((R, 1), lambda i: (i, 0)),
        pl.BlockSpec((R, 1), lambda i: (i, 0)),
        pl.BlockSpec((R, D), lambda i: (i, 0)),
    ]
    if residual:
        in_specs.append(pl.BlockSpec((R, D), lambda i: (i, 0)))
    in_specs += [
        pl.BlockSpec((D, D), lambda i: (0, 0)),
        pl.BlockSpec((D, D), lambda i: (0, 0)),
        pl.BlockSpec((1, D), lambda i: (0, 0)),
    ]
    if residual:
        in_specs += [
            pl.BlockSpec((D, D), lambda i: (0, 0)),
            pl.BlockSpec((1, D), lambda i: (0, 0)),
        ]
    return pl.pallas_call(
        body,
        grid=(N_NODES // R,),
        in_specs=in_specs,
        out_specs=pl.BlockSpec((R, D), lambda i: (i, 0)),
        out_shape=jax.ShapeDtypeStruct((N_NODES, D), jnp.float32),
    )


_SC_AGG_DEG = _make_sc_aggregate(with_deg=True)
_SC_AGG = _make_sc_aggregate(with_deg=False)
_EPI1 = _make_epilogue(relu=True, residual=False, sigmoid4=False)
_EPI2 = _make_epilogue(relu=True, residual=True, sigmoid4=False)
_EPI3 = _make_epilogue(relu=False, residual=True, sigmoid4=True)


def kernel(z, edge_index, Wl1, Wr1, b1, Wl2, Wr2, b2, Wres1, bres1,
           Wlf, Wrf, bf, Wres2, bres2):
    src = edge_index[0].astype(jnp.int32)
    dst = edge_index[1].astype(jnp.int32)

    agg1, deg_flat = _SC_AGG_DEG(z, src, dst)
    deg2 = deg_flat.reshape(NC, N_NODES)
    d0 = deg2[0].reshape(N_NODES, 1)
    d1 = deg2[1].reshape(N_NODES, 1)

    h1 = _EPI1(agg1, d0, d1, z, Wl1, Wr1, b1.reshape(1, D))
    agg2 = _SC_AGG(h1, src, dst)
    h2 = _EPI2(agg2, d0, d1, h1, z, Wl2, Wr2, b2.reshape(1, D),
               Wres1, bres1.reshape(1, D))
    agg3 = _SC_AGG(h2, src, dst)
    out = _EPI3(agg3, d0, d1, h2, z, Wlf, Wrf, bf.reshape(1, D),
                Wres2, bres2.reshape(1, D))
    return out
